# baseline (speedup 1.0000x reference)
import functools

import jax
import jax.numpy as jnp
from jax import lax
from jax.experimental import pallas as pl
from jax.experimental.pallas import tpu as pltpu
from jax.experimental.pallas import tpu_sc as plsc

CUTOFF = 0.8
KE = 14.399645351950548
E = 6400000
N_TAB = 100000
TABW = N_TAB // 2
NC, NS, L = 2, 16, 16
NW = NC * NS
EPW = E // NW
CHUNK = 2000
NCHUNK = EPW // CHUNK
UNROLL = 25
assert CHUNK % (L * UNROLL) == 0
assert NCHUNK % 2 == 0
assert EPW % CHUNK == 0
assert 2 * TABW + 2 * 4 * CHUNK <= 131000


def _lookup(tab_ref, idx):
    w = plsc.load_gather(tab_ref, [lax.shift_right_logical(idx, 1)])
    even_bits = lax.shift_left(w, 16)
    odd_bits = lax.bitwise_and(w, jnp.int32(-65536))
    bits = jnp.where(lax.bitwise_and(idx, 1) == 0, even_bits, odd_bits)
    return plsc.bitcast(bits, jnp.float32)


def _body(dist_hbm, qu_hbm, qv_hbm, idxu_hbm, idxv_hbm, out_hbm,
          qu_tab, qv_tab, bufs, load_sems, store_sems):
    wid = lax.axis_index("c") * NS + lax.axis_index("s")
    pltpu.sync_copy(qu_hbm, qu_tab)
    pltpu.sync_copy(qv_hbm, qv_tab)
    base0 = pl.multiple_of(wid * EPW, 8)

    def start_loads(s, ci):
        base = pl.multiple_of(base0 + ci * CHUNK, 8)
        idxu_v, idxv_v, dist_v, _ = bufs[s]
        pltpu.async_copy(idxu_hbm.at[pl.ds(base, CHUNK)], idxu_v, load_sems[s])
        pltpu.async_copy(idxv_hbm.at[pl.ds(base, CHUNK)], idxv_v, load_sems[s])
        pltpu.async_copy(dist_hbm.at[pl.ds(base, CHUNK)], dist_v, load_sems[s])

    def wait_loads(s):
        idxu_v, idxv_v, dist_v, _ = bufs[s]
        pltpu.make_async_copy(idxu_hbm.at[pl.ds(0, CHUNK)], idxu_v,
                              load_sems[s]).wait()
        pltpu.make_async_copy(idxv_hbm.at[pl.ds(0, CHUNK)], idxv_v,
                              load_sems[s]).wait()
        pltpu.make_async_copy(dist_hbm.at[pl.ds(0, CHUNK)], dist_v,
                              load_sems[s]).wait()

    def start_store(s, ci):
        base = pl.multiple_of(base0 + ci * CHUNK, 8)
        out_v = bufs[s][3]
        pltpu.async_copy(out_v, out_hbm.at[pl.ds(base, CHUNK)], store_sems[s])

    def wait_store(s):
        out_v = bufs[s][3]
        pltpu.make_async_copy(out_v, out_hbm.at[pl.ds(0, CHUNK)],
                              store_sems[s]).wait()

    def compute(s):
        idxu_v, idxv_v, dist_v, out_v = bufs[s]

        @plsc.parallel_loop(0, CHUNK // L, 1, unroll=UNROLL)
        def _(j):
            sl = pl.ds(pl.multiple_of(j * L, 8), L)
            u = idxu_v[sl]
            v = idxv_v[sl]
            d = dist_v[sl]
            qu = _lookup(qu_tab, u)
            qv = _lookup(qv_tab, v)
            e = (KE * qu) * qv / d
            out_v[sl] = jnp.where(d <= CUTOFF, e, 0.0)

    start_loads(0, 0)

    def pair_body(p, carry):
        c0 = 2 * p
        start_loads(1, c0 + 1)
        wait_loads(0)

        @pl.when(p > 0)
        def _():
            wait_store(0)

        compute(0)
        start_store(0, c0)

        @pl.when(c0 + 2 < NCHUNK)
        def _():
            start_loads(0, c0 + 2)

        wait_loads(1)

        @pl.when(p > 0)
        def _():
            wait_store(1)

        compute(1)
        start_store(1, c0 + 1)
        return carry

    lax.fori_loop(0, NCHUNK // 2, pair_body, 0)
    wait_store(0)
    wait_store(1)


@functools.partial(
    pl.kernel,
    out_type=jax.ShapeDtypeStruct((E,), jnp.float32),
    mesh=plsc.VectorSubcoreMesh(core_axis_name="c", subcore_axis_name="s"),
    compiler_params=pltpu.CompilerParams(needs_layout_passes=False),
    scratch_types=[
        pltpu.VMEM((TABW,), jnp.int32),
        pltpu.VMEM((TABW,), jnp.int32),
        [[pltpu.VMEM((CHUNK,), jnp.int32),
          pltpu.VMEM((CHUNK,), jnp.int32),
          pltpu.VMEM((CHUNK,), jnp.float32),
          pltpu.VMEM((CHUNK,), jnp.float32)] for _ in range(2)],
        [pltpu.SemaphoreType.DMA for _ in range(2)],
        [pltpu.SemaphoreType.DMA for _ in range(2)],
    ],
)
def _sc_coulomb(dist, qu_p, qv_p, idxu, idxv, out,
                qu_tab, qv_tab, bufs, load_sems, store_sems):
    _body(dist, qu_p, qv_p, idxu, idxv, out,
          qu_tab, qv_tab, bufs, load_sems, store_sems)


def _pack(q):
    return lax.bitcast_convert_type(
        q.astype(jnp.bfloat16).reshape(-1, 2), jnp.int32)


def kernel(mlmm_distances_uv, atomic_charges, mm_atomic_charges,
           mlmm_idx_u, mlmm_idx_v):
    qu_p = _pack(atomic_charges)
    qv_p = _pack(mm_atomic_charges)
    idxu = mlmm_idx_u.astype(jnp.int32)
    idxv = mlmm_idx_v.astype(jnp.int32)
    return _sc_coulomb(mlmm_distances_uv, qu_p, qv_p, idxu, idxv)


# speedup vs baseline: 1.2945x; 1.2945x over previous
import functools

import jax
import jax.numpy as jnp
from jax import lax
from jax.experimental import pallas as pl
from jax.experimental.pallas import tpu as pltpu
from jax.experimental.pallas import tpu_sc as plsc

CUTOFF = 0.8
KE = 14.399645351950548
E = 6400000
N_TAB = 100000
TABW = N_TAB // 2
NC, NS, L = 2, 16, 16
NW = NC * NS
EPW = E // NW
CHUNK = 2000
NCHUNK = EPW // CHUNK
UNROLL = 5
assert CHUNK % (L * UNROLL) == 0
assert NCHUNK % 2 == 0
assert EPW % CHUNK == 0
assert 2 * TABW + 2 * 4 * CHUNK <= 131000


def _lookup(tab_ref, idx):
    w = plsc.load_gather(tab_ref, [lax.shift_right_logical(idx, 1)])
    even_bits = lax.shift_left(w, 16)
    odd_bits = lax.bitwise_and(w, jnp.int32(-65536))
    bits = jnp.where(lax.bitwise_and(idx, 1) == 0, even_bits, odd_bits)
    return plsc.bitcast(bits, jnp.float32)


def _body(dist_hbm, qu_hbm, qv_hbm, idxu_hbm, idxv_hbm, out_hbm,
          qu_tab, qv_tab, bufs, load_sems, store_sems):
    wid = lax.axis_index("c") * NS + lax.axis_index("s")
    pltpu.sync_copy(qu_hbm, qu_tab)
    pltpu.sync_copy(qv_hbm, qv_tab)
    base0 = pl.multiple_of(wid * EPW, 8)

    def start_loads(s, ci):
        base = pl.multiple_of(base0 + ci * CHUNK, 8)
        idxu_v, idxv_v, dist_v, _ = bufs[s]
        pltpu.async_copy(idxu_hbm.at[pl.ds(base, CHUNK)], idxu_v, load_sems[s])
        pltpu.async_copy(idxv_hbm.at[pl.ds(base, CHUNK)], idxv_v, load_sems[s])
        pltpu.async_copy(dist_hbm.at[pl.ds(base, CHUNK)], dist_v, load_sems[s])

    def wait_loads(s):
        idxu_v, idxv_v, dist_v, _ = bufs[s]
        pltpu.make_async_copy(idxu_hbm.at[pl.ds(0, CHUNK)], idxu_v,
                              load_sems[s]).wait()
        pltpu.make_async_copy(idxv_hbm.at[pl.ds(0, CHUNK)], idxv_v,
                              load_sems[s]).wait()
        pltpu.make_async_copy(dist_hbm.at[pl.ds(0, CHUNK)], dist_v,
                              load_sems[s]).wait()

    def start_store(s, ci):
        base = pl.multiple_of(base0 + ci * CHUNK, 8)
        out_v = bufs[s][3]
        pltpu.async_copy(out_v, out_hbm.at[pl.ds(base, CHUNK)], store_sems[s])

    def wait_store(s):
        out_v = bufs[s][3]
        pltpu.make_async_copy(out_v, out_hbm.at[pl.ds(0, CHUNK)],
                              store_sems[s]).wait()

    def compute(s):
        idxu_v, idxv_v, dist_v, out_v = bufs[s]

        @plsc.parallel_loop(0, CHUNK // L, 1, unroll=UNROLL)
        def _(j):
            sl = pl.ds(pl.multiple_of(j * L, 8), L)
            u = idxu_v[sl]
            v = idxv_v[sl]
            d = dist_v[sl]
            qu = _lookup(qu_tab, u)
            qv = _lookup(qv_tab, v)
            e = (KE * qu) * qv / d
            out_v[sl] = jnp.where(d <= CUTOFF, e, 0.0)

    start_loads(0, 0)

    def pair_body(p, carry):
        c0 = 2 * p
        start_loads(1, c0 + 1)
        wait_loads(0)

        @pl.when(p > 0)
        def _():
            wait_store(0)

        compute(0)
        start_store(0, c0)

        @pl.when(c0 + 2 < NCHUNK)
        def _():
            start_loads(0, c0 + 2)

        wait_loads(1)

        @pl.when(p > 0)
        def _():
            wait_store(1)

        compute(1)
        start_store(1, c0 + 1)
        return carry

    lax.fori_loop(0, NCHUNK // 2, pair_body, 0)
    wait_store(0)
    wait_store(1)


@functools.partial(
    pl.kernel,
    out_type=jax.ShapeDtypeStruct((E,), jnp.float32),
    mesh=plsc.VectorSubcoreMesh(core_axis_name="c", subcore_axis_name="s"),
    compiler_params=pltpu.CompilerParams(needs_layout_passes=False),
    scratch_types=[
        pltpu.VMEM((TABW,), jnp.int32),
        pltpu.VMEM((TABW,), jnp.int32),
        [[pltpu.VMEM((CHUNK,), jnp.int32),
          pltpu.VMEM((CHUNK,), jnp.int32),
          pltpu.VMEM((CHUNK,), jnp.float32),
          pltpu.VMEM((CHUNK,), jnp.float32)] for _ in range(2)],
        [pltpu.SemaphoreType.DMA for _ in range(2)],
        [pltpu.SemaphoreType.DMA for _ in range(2)],
    ],
)
def _sc_coulomb(dist, qu_p, qv_p, idxu, idxv, out,
                qu_tab, qv_tab, bufs, load_sems, store_sems):
    _body(dist, qu_p, qv_p, idxu, idxv, out,
          qu_tab, qv_tab, bufs, load_sems, store_sems)


def _pack(q):
    return lax.bitcast_convert_type(
        q.astype(jnp.bfloat16).reshape(-1, 2), jnp.int32)


def kernel(mlmm_distances_uv, atomic_charges, mm_atomic_charges,
           mlmm_idx_u, mlmm_idx_v):
    qu_p = _pack(atomic_charges)
    qv_p = _pack(mm_atomic_charges)
    idxu = mlmm_idx_u.astype(jnp.int32)
    idxv = mlmm_idx_v.astype(jnp.int32)
    return _sc_coulomb(mlmm_distances_uv, qu_p, qv_p, idxu, idxv)


# speedup vs baseline: 1.3845x; 1.0695x over previous
import functools

import jax
import jax.numpy as jnp
from jax import lax
from jax.experimental import pallas as pl
from jax.experimental.pallas import tpu as pltpu
from jax.experimental.pallas import tpu_sc as plsc

CUTOFF = 0.8
KE = 14.399645351950548
E = 6400000
N_TAB = 100000
TABW = N_TAB // 2
NC, NS, L = 2, 16, 16
NW = NC * NS
EPW = E // NW
CHUNK = 3200
NCHUNK = E // CHUNK
UNROLL = 5
assert CHUNK % (L * UNROLL) == 0
assert E % CHUNK == 0
assert 2 * TABW + 2 * 4 * CHUNK <= 131000


def _lookup(tab_ref, idx):
    w = plsc.load_gather(tab_ref, [lax.shift_right_logical(idx, 1)])
    even_bits = lax.shift_left(w, 16)
    odd_bits = lax.bitwise_and(w, jnp.int32(-65536))
    bits = jnp.where(lax.bitwise_and(idx, 1) == 0, even_bits, odd_bits)
    return plsc.bitcast(bits, jnp.float32)


def _body(dist_hbm, qu_hbm, qv_hbm, idxu_hbm, idxv_hbm, out_hbm,
          qu_tab, qv_tab, bufs, load_sems, store_sems):
    wid = lax.axis_index("c") * NS + lax.axis_index("s")
    pltpu.sync_copy(qu_hbm, qu_tab)
    pltpu.sync_copy(qv_hbm, qv_tab)
    nk = (NCHUNK - 1 - wid) // NW + 1

    def start_loads(s, k):
        base = pl.multiple_of((wid + NW * k) * CHUNK, 8)
        idxu_v, idxv_v, dist_v, _ = bufs[s]
        pltpu.async_copy(idxu_hbm.at[pl.ds(base, CHUNK)], idxu_v, load_sems[s])
        pltpu.async_copy(idxv_hbm.at[pl.ds(base, CHUNK)], idxv_v, load_sems[s])
        pltpu.async_copy(dist_hbm.at[pl.ds(base, CHUNK)], dist_v, load_sems[s])

    def wait_loads(s):
        idxu_v, idxv_v, dist_v, _ = bufs[s]
        pltpu.make_async_copy(idxu_hbm.at[pl.ds(0, CHUNK)], idxu_v,
                              load_sems[s]).wait()
        pltpu.make_async_copy(idxv_hbm.at[pl.ds(0, CHUNK)], idxv_v,
                              load_sems[s]).wait()
        pltpu.make_async_copy(dist_hbm.at[pl.ds(0, CHUNK)], dist_v,
                              load_sems[s]).wait()

    def start_store(s, k):
        base = pl.multiple_of((wid + NW * k) * CHUNK, 8)
        out_v = bufs[s][3]
        pltpu.async_copy(out_v, out_hbm.at[pl.ds(base, CHUNK)], store_sems[s])

    def wait_store(s):
        out_v = bufs[s][3]
        pltpu.make_async_copy(out_v, out_hbm.at[pl.ds(0, CHUNK)],
                              store_sems[s]).wait()

    def compute(s):
        idxu_v, idxv_v, dist_v, out_v = bufs[s]

        @plsc.parallel_loop(0, CHUNK // L, 1, unroll=UNROLL)
        def _(j):
            sl = pl.ds(pl.multiple_of(j * L, 8), L)
            u = idxu_v[sl]
            v = idxv_v[sl]
            d = dist_v[sl]
            qu = _lookup(qu_tab, u)
            qv = _lookup(qv_tab, v)
            e = (KE * qu) * qv / d
            out_v[sl] = jnp.where(d <= CUTOFF, e, 0.0)

    start_loads(0, 0)

    def pair_body(p, carry):
        k0 = 2 * p
        start_loads(1, k0 + 1)
        wait_loads(0)

        @pl.when(p > 0)
        def _():
            wait_store(0)

        compute(0)
        start_store(0, k0)

        @pl.when(k0 + 2 < nk)
        def _():
            start_loads(0, k0 + 2)

        wait_loads(1)

        @pl.when(p > 0)
        def _():
            wait_store(1)

        compute(1)
        start_store(1, k0 + 1)
        return carry

    lax.fori_loop(0, nk // 2, pair_body, 0)

    @pl.when(nk % 2 == 1)
    def _():
        wait_loads(0)
        wait_store(0)
        compute(0)
        start_store(0, nk - 1)

    wait_store(0)
    wait_store(1)


@functools.partial(
    pl.kernel,
    out_type=jax.ShapeDtypeStruct((E,), jnp.float32),
    mesh=plsc.VectorSubcoreMesh(core_axis_name="c", subcore_axis_name="s"),
    compiler_params=pltpu.CompilerParams(needs_layout_passes=False),
    scratch_types=[
        pltpu.VMEM((TABW,), jnp.int32),
        pltpu.VMEM((TABW,), jnp.int32),
        [[pltpu.VMEM((CHUNK,), jnp.int32),
          pltpu.VMEM((CHUNK,), jnp.int32),
          pltpu.VMEM((CHUNK,), jnp.float32),
          pltpu.VMEM((CHUNK,), jnp.float32)] for _ in range(2)],
        [pltpu.SemaphoreType.DMA for _ in range(2)],
        [pltpu.SemaphoreType.DMA for _ in range(2)],
    ],
)
def _sc_coulomb(dist, qu_p, qv_p, idxu, idxv, out,
                qu_tab, qv_tab, bufs, load_sems, store_sems):
    _body(dist, qu_p, qv_p, idxu, idxv, out,
          qu_tab, qv_tab, bufs, load_sems, store_sems)


def _pack(q):
    return lax.bitcast_convert_type(
        q.astype(jnp.bfloat16).reshape(-1, 2), jnp.int32)


def kernel(mlmm_distances_uv, atomic_charges, mm_atomic_charges,
           mlmm_idx_u, mlmm_idx_v):
    qu_p = _pack(atomic_charges)
    qv_p = _pack(mm_atomic_charges)
    idxu = mlmm_idx_u.astype(jnp.int32)
    idxv = mlmm_idx_v.astype(jnp.int32)
    return _sc_coulomb(mlmm_distances_uv, qu_p, qv_p, idxu, idxv)


# speedup vs baseline: 1.4021x; 1.0127x over previous
import functools

import jax
import jax.numpy as jnp
from jax import lax
from jax.experimental import pallas as pl
from jax.experimental.pallas import tpu as pltpu
from jax.experimental.pallas import tpu_sc as plsc

CUTOFF = 0.8
KE = 14.399645351950548
E = 6400000
N_TAB = 100000
TABW = N_TAB // 2
NC, NS, L = 2, 16, 16
NW = NC * NS
EPW = E // NW
CHUNK = 3200
NCHUNK = E // CHUNK
UNROLL = 5
assert CHUNK % (L * UNROLL) == 0
assert E % CHUNK == 0
assert 2 * TABW + 2 * 4 * CHUNK <= 131000


def _lookup(tab_ref, idx):
    w = plsc.load_gather(tab_ref, [lax.shift_right_logical(idx, 1)])
    sh = lax.shift_left(lax.bitwise_and(idx, 1), 4)
    bits = lax.shift_left(lax.shift_right_logical(w, sh), 16)
    return plsc.bitcast(bits, jnp.float32)


def _body(dist_hbm, qu_hbm, qv_hbm, idxu_hbm, idxv_hbm, out_hbm,
          qu_tab, qv_tab, bufs, load_sems, store_sems):
    wid = lax.axis_index("c") * NS + lax.axis_index("s")
    pltpu.sync_copy(qu_hbm, qu_tab)
    pltpu.sync_copy(qv_hbm, qv_tab)
    nk = (NCHUNK - 1 - wid) // NW + 1

    def start_loads(s, k):
        base = pl.multiple_of((wid + NW * k) * CHUNK, 8)
        idxu_v, idxv_v, dist_v, _ = bufs[s]
        pltpu.async_copy(idxu_hbm.at[pl.ds(base, CHUNK)], idxu_v, load_sems[s])
        pltpu.async_copy(idxv_hbm.at[pl.ds(base, CHUNK)], idxv_v, load_sems[s])
        pltpu.async_copy(dist_hbm.at[pl.ds(base, CHUNK)], dist_v, load_sems[s])

    def wait_loads(s):
        idxu_v, idxv_v, dist_v, _ = bufs[s]
        pltpu.make_async_copy(idxu_hbm.at[pl.ds(0, CHUNK)], idxu_v,
                              load_sems[s]).wait()
        pltpu.make_async_copy(idxv_hbm.at[pl.ds(0, CHUNK)], idxv_v,
                              load_sems[s]).wait()
        pltpu.make_async_copy(dist_hbm.at[pl.ds(0, CHUNK)], dist_v,
                              load_sems[s]).wait()

    def start_store(s, k):
        base = pl.multiple_of((wid + NW * k) * CHUNK, 8)
        out_v = bufs[s][3]
        pltpu.async_copy(out_v, out_hbm.at[pl.ds(base, CHUNK)], store_sems[s])

    def wait_store(s):
        out_v = bufs[s][3]
        pltpu.make_async_copy(out_v, out_hbm.at[pl.ds(0, CHUNK)],
                              store_sems[s]).wait()

    def compute(s):
        idxu_v, idxv_v, dist_v, out_v = bufs[s]

        @plsc.parallel_loop(0, CHUNK // L, 1, unroll=UNROLL)
        def _(j):
            sl = pl.ds(pl.multiple_of(j * L, 8), L)
            u = idxu_v[sl]
            v = idxv_v[sl]
            d = dist_v[sl]
            qu = _lookup(qu_tab, u)
            qv = _lookup(qv_tab, v)
            e = qu * qv / d
            out_v[sl] = jnp.where(d <= CUTOFF, e, 0.0)

    start_loads(0, 0)

    def pair_body(p, carry):
        k0 = 2 * p
        start_loads(1, k0 + 1)
        wait_loads(0)

        @pl.when(p > 0)
        def _():
            wait_store(0)

        compute(0)
        start_store(0, k0)

        @pl.when(k0 + 2 < nk)
        def _():
            start_loads(0, k0 + 2)

        wait_loads(1)

        @pl.when(p > 0)
        def _():
            wait_store(1)

        compute(1)
        start_store(1, k0 + 1)
        return carry

    lax.fori_loop(0, nk // 2, pair_body, 0)

    @pl.when(nk % 2 == 1)
    def _():
        wait_loads(0)
        wait_store(0)
        compute(0)
        start_store(0, nk - 1)

    wait_store(0)
    wait_store(1)


@functools.partial(
    pl.kernel,
    out_type=jax.ShapeDtypeStruct((E,), jnp.float32),
    mesh=plsc.VectorSubcoreMesh(core_axis_name="c", subcore_axis_name="s"),
    compiler_params=pltpu.CompilerParams(needs_layout_passes=False),
    scratch_types=[
        pltpu.VMEM((TABW,), jnp.int32),
        pltpu.VMEM((TABW,), jnp.int32),
        [[pltpu.VMEM((CHUNK,), jnp.int32),
          pltpu.VMEM((CHUNK,), jnp.int32),
          pltpu.VMEM((CHUNK,), jnp.float32),
          pltpu.VMEM((CHUNK,), jnp.float32)] for _ in range(2)],
        [pltpu.SemaphoreType.DMA for _ in range(2)],
        [pltpu.SemaphoreType.DMA for _ in range(2)],
    ],
)
def _sc_coulomb(dist, qu_p, qv_p, idxu, idxv, out,
                qu_tab, qv_tab, bufs, load_sems, store_sems):
    _body(dist, qu_p, qv_p, idxu, idxv, out,
          qu_tab, qv_tab, bufs, load_sems, store_sems)


def _pack(q):
    return lax.bitcast_convert_type(
        q.astype(jnp.bfloat16).reshape(-1, 2), jnp.int32)


def kernel(mlmm_distances_uv, atomic_charges, mm_atomic_charges,
           mlmm_idx_u, mlmm_idx_v):
    qu_p = _pack(KE * atomic_charges)
    qv_p = _pack(mm_atomic_charges)
    idxu = mlmm_idx_u.astype(jnp.int32)
    idxv = mlmm_idx_v.astype(jnp.int32)
    return _sc_coulomb(mlmm_distances_uv, qu_p, qv_p, idxu, idxv)


# speedup vs baseline: 2.6759x; 1.9085x over previous
import functools

import jax
import jax.numpy as jnp
from jax import lax
from jax.experimental import pallas as pl
from jax.experimental.pallas import tpu as pltpu
from jax.experimental.pallas import tpu_sc as plsc

CUTOFF = 0.8
KE = 14.399645351950548
E = 6400000
N_TAB = 100000
TABW = N_TAB // 2
NC, NS, L = 2, 16, 16
NW = NC * NS
EPW = E // NW
CHUNK = 3200
NCHUNK = E // CHUNK
UNROLL = 5
assert CHUNK % (L * UNROLL) == 0
assert E % CHUNK == 0
assert 2 * TABW + 2 * 4 * CHUNK <= 131000


def _lookup(tab_ref, idx):
    d_ = idx - TABW
    word = plsc.bitcast(
        jnp.minimum(plsc.bitcast(idx, jnp.uint32),
                    plsc.bitcast(d_, jnp.uint32)),
        jnp.int32)
    w = plsc.load_gather(tab_ref, [word])
    sh = jnp.where(idx >= TABW, 16, 0)
    bits = lax.shift_left(lax.shift_right_logical(w, sh), 16)
    return plsc.bitcast(bits, jnp.float32)


def _body(dist_hbm, qu_hbm, qv_hbm, idxu_hbm, idxv_hbm, out_hbm,
          qu_tab, qv_tab, bufs, load_sems, store_sems):
    wid = lax.axis_index("c") * NS + lax.axis_index("s")
    pltpu.sync_copy(qu_hbm, qu_tab)
    pltpu.sync_copy(qv_hbm, qv_tab)
    nk = (NCHUNK - 1 - wid) // NW + 1

    def start_loads(s, k):
        base = pl.multiple_of((wid + NW * k) * CHUNK, 8)
        idxu_v, idxv_v, dist_v, _ = bufs[s]
        pltpu.async_copy(idxu_hbm.at[pl.ds(base, CHUNK)], idxu_v, load_sems[s])
        pltpu.async_copy(idxv_hbm.at[pl.ds(base, CHUNK)], idxv_v, load_sems[s])
        pltpu.async_copy(dist_hbm.at[pl.ds(base, CHUNK)], dist_v, load_sems[s])

    def wait_loads(s):
        idxu_v, idxv_v, dist_v, _ = bufs[s]
        pltpu.make_async_copy(idxu_hbm.at[pl.ds(0, CHUNK)], idxu_v,
                              load_sems[s]).wait()
        pltpu.make_async_copy(idxv_hbm.at[pl.ds(0, CHUNK)], idxv_v,
                              load_sems[s]).wait()
        pltpu.make_async_copy(dist_hbm.at[pl.ds(0, CHUNK)], dist_v,
                              load_sems[s]).wait()

    def start_store(s, k):
        base = pl.multiple_of((wid + NW * k) * CHUNK, 8)
        out_v = bufs[s][3]
        pltpu.async_copy(out_v, out_hbm.at[pl.ds(base, CHUNK)], store_sems[s])

    def wait_store(s):
        out_v = bufs[s][3]
        pltpu.make_async_copy(out_v, out_hbm.at[pl.ds(0, CHUNK)],
                              store_sems[s]).wait()

    def compute(s):
        idxu_v, idxv_v, dist_v, out_v = bufs[s]

        @plsc.parallel_loop(0, CHUNK // L, 1, unroll=UNROLL)
        def _(j):
            sl = pl.ds(pl.multiple_of(j * L, 8), L)
            u = idxu_v[sl]
            v = idxv_v[sl]
            d = dist_v[sl]
            qu = _lookup(qu_tab, u)
            qv = _lookup(qv_tab, v)
            e = qu * qv / d
            out_v[sl] = jnp.where(d <= CUTOFF, e, 0.0)

    start_loads(0, 0)

    def pair_body(p, carry):
        k0 = 2 * p
        start_loads(1, k0 + 1)
        wait_loads(0)

        @pl.when(p > 0)
        def _():
            wait_store(0)

        compute(0)
        start_store(0, k0)

        @pl.when(k0 + 2 < nk)
        def _():
            start_loads(0, k0 + 2)

        wait_loads(1)

        @pl.when(p > 0)
        def _():
            wait_store(1)

        compute(1)
        start_store(1, k0 + 1)
        return carry

    lax.fori_loop(0, nk // 2, pair_body, 0)

    @pl.when(nk % 2 == 1)
    def _():
        wait_loads(0)
        wait_store(0)
        compute(0)
        start_store(0, nk - 1)

    wait_store(0)
    wait_store(1)


@functools.partial(
    pl.kernel,
    out_type=jax.ShapeDtypeStruct((E,), jnp.float32),
    mesh=plsc.VectorSubcoreMesh(core_axis_name="c", subcore_axis_name="s"),
    compiler_params=pltpu.CompilerParams(needs_layout_passes=False),
    scratch_types=[
        pltpu.VMEM((TABW,), jnp.int32),
        pltpu.VMEM((TABW,), jnp.int32),
        [[pltpu.VMEM((CHUNK,), jnp.int32),
          pltpu.VMEM((CHUNK,), jnp.int32),
          pltpu.VMEM((CHUNK,), jnp.float32),
          pltpu.VMEM((CHUNK,), jnp.float32)] for _ in range(2)],
        [pltpu.SemaphoreType.DMA for _ in range(2)],
        [pltpu.SemaphoreType.DMA for _ in range(2)],
    ],
)
def _sc_coulomb(dist, qu_p, qv_p, idxu, idxv, out,
                qu_tab, qv_tab, bufs, load_sems, store_sems):
    _body(dist, qu_p, qv_p, idxu, idxv, out,
          qu_tab, qv_tab, bufs, load_sems, store_sems)


def _pack(q):
    b = lax.bitcast_convert_type(q.astype(jnp.float32), jnp.int32)
    rb = b + 0x7FFF + lax.bitwise_and(lax.shift_right_logical(b, 16), 1)
    return (lax.shift_right_logical(rb[:TABW], 16)
            | (rb[TABW:] & jnp.int32(-65536)))


def kernel(mlmm_distances_uv, atomic_charges, mm_atomic_charges,
           mlmm_idx_u, mlmm_idx_v):
    qu_p = _pack(KE * atomic_charges)
    qv_p = _pack(mm_atomic_charges)
    idxu = mlmm_idx_u.astype(jnp.int32)
    idxv = mlmm_idx_v.astype(jnp.int32)
    return _sc_coulomb(mlmm_distances_uv, qu_p, qv_p, idxu, idxv)


# speedup vs baseline: 2.7154x; 1.0148x over previous
import functools

import jax
import jax.numpy as jnp
from jax import lax
from jax.experimental import pallas as pl
from jax.experimental.pallas import tpu as pltpu
from jax.experimental.pallas import tpu_sc as plsc

CUTOFF = 0.8
KE = 14.399645351950548
E = 6400000
N_TAB = 100000
TABW = N_TAB // 2
NC, NS, L = 2, 16, 16
NW = NC * NS
EPW = E // NW
CHUNK = 3200
NCHUNK = E // CHUNK
UNROLL = 10
assert CHUNK % (L * UNROLL) == 0
assert E % CHUNK == 0
assert 2 * TABW + 2 * 4 * CHUNK <= 131000


def _lookup(tab_ref, idx):
    d_ = idx - TABW
    word = plsc.bitcast(
        jnp.minimum(plsc.bitcast(idx, jnp.uint32),
                    plsc.bitcast(d_, jnp.uint32)),
        jnp.int32)
    w = plsc.load_gather(tab_ref, [word])
    sh = jnp.where(idx >= TABW, 16, 0)
    bits = lax.shift_left(lax.shift_right_logical(w, sh), 16)
    return plsc.bitcast(bits, jnp.float32)


def _body(dist_hbm, qu_hbm, qv_hbm, idxu_hbm, idxv_hbm, out_hbm,
          qu_tab, qv_tab, bufs, load_sems, store_sems):
    wid = lax.axis_index("c") * NS + lax.axis_index("s")
    pltpu.sync_copy(qu_hbm, qu_tab)
    pltpu.sync_copy(qv_hbm, qv_tab)
    nk = (NCHUNK - 1 - wid) // NW + 1

    def start_loads(s, k):
        base = pl.multiple_of((wid + NW * k) * CHUNK, 8)
        idxu_v, idxv_v, dist_v, _ = bufs[s]
        pltpu.async_copy(idxu_hbm.at[pl.ds(base, CHUNK)], idxu_v, load_sems[s])
        pltpu.async_copy(idxv_hbm.at[pl.ds(base, CHUNK)], idxv_v, load_sems[s])
        pltpu.async_copy(dist_hbm.at[pl.ds(base, CHUNK)], dist_v, load_sems[s])

    def wait_loads(s):
        idxu_v, idxv_v, dist_v, _ = bufs[s]
        pltpu.make_async_copy(idxu_hbm.at[pl.ds(0, CHUNK)], idxu_v,
                              load_sems[s]).wait()
        pltpu.make_async_copy(idxv_hbm.at[pl.ds(0, CHUNK)], idxv_v,
                              load_sems[s]).wait()
        pltpu.make_async_copy(dist_hbm.at[pl.ds(0, CHUNK)], dist_v,
                              load_sems[s]).wait()

    def start_store(s, k):
        base = pl.multiple_of((wid + NW * k) * CHUNK, 8)
        out_v = bufs[s][3]
        pltpu.async_copy(out_v, out_hbm.at[pl.ds(base, CHUNK)], store_sems[s])

    def wait_store(s):
        out_v = bufs[s][3]
        pltpu.make_async_copy(out_v, out_hbm.at[pl.ds(0, CHUNK)],
                              store_sems[s]).wait()

    def compute(s):
        idxu_v, idxv_v, dist_v, out_v = bufs[s]

        @plsc.parallel_loop(0, CHUNK // L, 1, unroll=UNROLL)
        def _(j):
            sl = pl.ds(pl.multiple_of(j * L, 8), L)
            u = idxu_v[sl]
            v = idxv_v[sl]
            d = dist_v[sl]
            qu = _lookup(qu_tab, u)
            qv = _lookup(qv_tab, v)
            e = qu * qv / d
            out_v[sl] = jnp.where(d <= CUTOFF, e, 0.0)

    start_loads(0, 0)

    def pair_body(p, carry):
        k0 = 2 * p
        start_loads(1, k0 + 1)
        wait_loads(0)

        @pl.when(p > 0)
        def _():
            wait_store(0)

        compute(0)
        start_store(0, k0)

        @pl.when(k0 + 2 < nk)
        def _():
            start_loads(0, k0 + 2)

        wait_loads(1)

        @pl.when(p > 0)
        def _():
            wait_store(1)

        compute(1)
        start_store(1, k0 + 1)
        return carry

    lax.fori_loop(0, nk // 2, pair_body, 0)

    @pl.when(nk % 2 == 1)
    def _():
        wait_loads(0)
        wait_store(0)
        compute(0)
        start_store(0, nk - 1)

    wait_store(0)
    wait_store(1)


@functools.partial(
    pl.kernel,
    out_type=jax.ShapeDtypeStruct((E,), jnp.float32),
    mesh=plsc.VectorSubcoreMesh(core_axis_name="c", subcore_axis_name="s"),
    compiler_params=pltpu.CompilerParams(needs_layout_passes=False),
    scratch_types=[
        pltpu.VMEM((TABW,), jnp.int32),
        pltpu.VMEM((TABW,), jnp.int32),
        [[pltpu.VMEM((CHUNK,), jnp.int32),
          pltpu.VMEM((CHUNK,), jnp.int32),
          pltpu.VMEM((CHUNK,), jnp.float32),
          pltpu.VMEM((CHUNK,), jnp.float32)] for _ in range(2)],
        [pltpu.SemaphoreType.DMA for _ in range(2)],
        [pltpu.SemaphoreType.DMA for _ in range(2)],
    ],
)
def _sc_coulomb(dist, qu_p, qv_p, idxu, idxv, out,
                qu_tab, qv_tab, bufs, load_sems, store_sems):
    _body(dist, qu_p, qv_p, idxu, idxv, out,
          qu_tab, qv_tab, bufs, load_sems, store_sems)


def _pack(q):
    b = lax.bitcast_convert_type(q.astype(jnp.float32), jnp.int32)
    rb = b + 0x7FFF + lax.bitwise_and(lax.shift_right_logical(b, 16), 1)
    return (lax.shift_right_logical(rb[:TABW], 16)
            | (rb[TABW:] & jnp.int32(-65536)))


def kernel(mlmm_distances_uv, atomic_charges, mm_atomic_charges,
           mlmm_idx_u, mlmm_idx_v):
    qu_p = _pack(KE * atomic_charges)
    qv_p = _pack(mm_atomic_charges)
    idxu = mlmm_idx_u.astype(jnp.int32)
    idxv = mlmm_idx_v.astype(jnp.int32)
    return _sc_coulomb(mlmm_distances_uv, qu_p, qv_p, idxu, idxv)
